# baseline (speedup 1.0000x reference)
import functools

import jax
import jax.numpy as jnp
from jax import lax
from jax.experimental import pallas as pl
from jax.experimental.pallas import tpu as pltpu
from jax.experimental.pallas import tpu_sc as plsc

VOCAB = 100000
EMBED = 128
BATCH = 4096
SEQ = 50

NC = 2
NS = 16
NW = NC * NS
C = BATCH // NW
L = 16
NBUF = 6
K = 3
NG = SEQ // NBUF

_mesh = plsc.VectorSubcoreMesh(core_axis_name="c", subcore_axis_name="s")


@functools.partial(
    pl.kernel,
    out_type=jax.ShapeDtypeStruct((SEQ, BATCH, EMBED), jnp.float32),
    mesh=_mesh,
    compiler_params=pltpu.CompilerParams(needs_layout_passes=False),
    scratch_types=(
        [pltpu.VMEM((C, SEQ), jnp.int32)]
        + [pltpu.VMEM((C,), jnp.int32) for _ in range(NBUF)]
        + [pltpu.VMEM((C, EMBED), jnp.float32) for _ in range(NBUF)]
        + [pltpu.SemaphoreType.DMA for _ in range(2 * NBUF)]
    ),
)
def _gather_kernel(idx_hbm, table_hbm, out_hbm, idx_raw, *scr):
    idxb = scr[:NBUF]
    rows = scr[NBUF:2 * NBUF]
    gsem = scr[2 * NBUF:3 * NBUF]
    osem = scr[3 * NBUF:]
    wid = lax.axis_index("s") * NC + lax.axis_index("c")
    base = wid * C

    pltpu.sync_copy(idx_hbm.at[pl.ds(base, C)], idx_raw)

    lanes = lax.iota(jnp.int32, L)

    def extract_col(j, b):
        cols = jnp.full((L,), j, jnp.int32)
        for t in range(C // L):
            v = plsc.load_gather(idx_raw, [t * L + lanes, cols])
            idxb[b][pl.ds(t * L, L)] = v

    def start_gather(j, b):
        extract_col(j, b)
        pltpu.async_copy(table_hbm.at[idxb[b]], rows[b], gsem[b])

    def wait_gather(b):
        pltpu.make_async_copy(table_hbm.at[pl.ds(0, C)], rows[b], gsem[b]).wait()

    def start_out(j, b):
        pltpu.async_copy(rows[b], out_hbm.at[j, pl.ds(base, C)], osem[b])

    def wait_out(b):
        pltpu.make_async_copy(rows[b], out_hbm.at[0, pl.ds(base, C)], osem[b]).wait()

    def step(j_pat, j_dyn):
        b = j_pat % NBUF
        if j_pat >= NBUF:
            wait_out(b)
        start_gather(j_dyn, b)
        if j_pat >= K:
            bo = (j_pat - K) % NBUF
            wait_gather(bo)
            start_out(j_dyn - K, bo)

    for j in range(NBUF):
        step(j, j)

    def group(g, carry):
        for b in range(NBUF):
            step(NBUF + b, g * NBUF + b)
        return carry

    lax.fori_loop(1, NG, group, 0)

    for j in range(NG * NBUF, SEQ):
        step(NBUF + (j % NBUF), j)

    for t in range(K):
        i = SEQ - K + t
        wait_gather(i % NBUF)
        start_out(i, i % NBUF)
    for b in range(NBUF):
        wait_out(b)


def kernel(indices, table):
    out = _gather_kernel(indices.astype(jnp.int32), table)
    return jnp.transpose(out, (1, 0, 2))


# speedup vs baseline: 1.0141x; 1.0141x over previous
import functools

import jax
import jax.numpy as jnp
from jax import lax
from jax.experimental import pallas as pl
from jax.experimental.pallas import tpu as pltpu
from jax.experimental.pallas import tpu_sc as plsc

VOCAB = 100000
EMBED = 128
BATCH = 4096
SEQ = 50

NC = 2
NS = 16
NW = NC * NS
C = BATCH // NW
NBUF = 6
K = 3
NG = SEQ // NBUF

_mesh = plsc.VectorSubcoreMesh(core_axis_name="c", subcore_axis_name="s")


@functools.partial(
    pl.kernel,
    out_type=jax.ShapeDtypeStruct((SEQ, BATCH, EMBED), jnp.float32),
    mesh=_mesh,
    scratch_types=(
        [pltpu.VMEM((SEQ, C), jnp.int32)]
        + [pltpu.VMEM((C, EMBED), jnp.float32) for _ in range(NBUF)]
        + [pltpu.SemaphoreType.DMA for _ in range(2 * NBUF)]
    ),
)
def _gather_kernel(idx_hbm, table_hbm, out_hbm, idx_v, *scr):
    rows = scr[:NBUF]
    gsem = scr[NBUF:2 * NBUF]
    osem = scr[2 * NBUF:]
    wid = lax.axis_index("s") * NC + lax.axis_index("c")
    base = wid * C

    pltpu.sync_copy(idx_hbm.at[pl.ds(0, SEQ), pl.ds(base, C)], idx_v)

    def start_gather(j, b):
        pltpu.async_copy(table_hbm.at[idx_v.at[j]], rows[b], gsem[b])

    def wait_gather(b):
        pltpu.make_async_copy(table_hbm.at[pl.ds(0, C)], rows[b], gsem[b]).wait()

    def start_out(j, b):
        pltpu.async_copy(rows[b], out_hbm.at[j, pl.ds(base, C)], osem[b])

    def wait_out(b):
        pltpu.make_async_copy(rows[b], out_hbm.at[0, pl.ds(base, C)], osem[b]).wait()

    def step(j_pat, j_dyn):
        b = j_pat % NBUF
        if j_pat >= NBUF:
            wait_out(b)
        start_gather(j_dyn, b)
        if j_pat >= K:
            bo = (j_pat - K) % NBUF
            wait_gather(bo)
            start_out(j_dyn - K, bo)

    for j in range(NBUF):
        step(j, j)

    def group(g, carry):
        for b in range(NBUF):
            step(NBUF + b, g * NBUF + b)
        return carry

    lax.fori_loop(1, NG, group, 0)

    for j in range(NG * NBUF, SEQ):
        step(NBUF + (j % NBUF), j)

    for t in range(K):
        i = SEQ - K + t
        wait_gather(i % NBUF)
        start_out(i, i % NBUF)
    for b in range(NBUF):
        wait_out(b)


def kernel(indices, table):
    idx = jnp.transpose(indices.astype(jnp.int32))
    out = _gather_kernel(idx, table)
    return jnp.transpose(out, (1, 0, 2))
